# baseline (speedup 1.0000x reference)
import jax
import jax.numpy as jnp
from jax import lax
from jax.experimental import pallas as pl
from jax.experimental.pallas import tpu as pltpu
from jax.experimental.pallas import tpu_sc as plsc

E = 8
G = 2
EG = E // G
D = 768
F = 512
T = 2048
B = 128
NB_G = T // B + EG
NBT = G * NB_G
PG = NB_G * B
ROWS = G * PG
CHUNK = 256

NC, NS = 2, 16
NW = NC * NS
TPW = T // NW


def _routing_body(x_ref, rw_ref, pos_ref, w_ref, be_ref):
    x = x_ref[...]
    logits = jnp.dot(x, rw_ref[...], preferred_element_type=jnp.float32)
    m = jnp.max(logits, axis=-1, keepdims=True)
    ex = jnp.exp(logits - m)
    p = ex / jnp.sum(ex, axis=-1, keepdims=True)
    col = lax.broadcasted_iota(jnp.int32, (T, E), 1)
    g0 = col < EG
    neg = jnp.float32(-1.0)
    w0 = jnp.max(jnp.where(g0, p, neg), axis=-1, keepdims=True)
    w1 = jnp.max(jnp.where(g0, neg, p), axis=-1, keepdims=True)
    big = jnp.int32(E)
    e0 = jnp.min(jnp.where(g0 & (p == w0), col, big), axis=-1, keepdims=True)
    e1 = jnp.min(jnp.where((~g0) & (p == w1), col, big), axis=-1, keepdims=True)
    s = w0 + w1
    oh0 = (col == e0).astype(jnp.float32)
    oh1 = (col == e1).astype(jnp.float32)
    oh = oh0 + oh1

    r_ = lax.broadcasted_iota(jnp.int32, (CHUNK, CHUNK), 0)
    c_ = lax.broadcasted_iota(jnp.int32, (CHUNK, CHUNK), 1)
    tril = (r_ >= c_).astype(jnp.float32)
    carry = jnp.zeros((1, E), jnp.float32)
    rank_chunks = []
    for c in range(T // CHUNK):
        oh_c = oh[c * CHUNK:(c + 1) * CHUNK, :]
        cum_c = jnp.dot(tril, oh_c, preferred_element_type=jnp.float32)
        rank_chunks.append(cum_c + carry - oh_c)
        carry = carry + jnp.sum(oh_c, axis=0, keepdims=True)
    rank = jnp.concatenate(rank_chunks, axis=0)

    counts = carry
    pc = (jnp.floor(counts / B) +
          jnp.where(counts % B > 0, 1.0, 0.0)) * B
    gi = lax.broadcasted_iota(jnp.int32, (E, E), 0)
    gj = lax.broadcasted_iota(jnp.int32, (E, E), 1)
    same = gi // EG == gj // EG
    prefix = (same & (gi < gj)).astype(jnp.float32)
    off = jnp.dot(pc, prefix, preferred_element_type=jnp.float32)

    posf = jnp.sum(oh * (off + rank), axis=-1, keepdims=True)
    pos0 = jnp.sum(oh0 * (off + rank), axis=-1, keepdims=True)
    pos1 = posf - pos0 + PG
    pos2 = jnp.concatenate([pos0, pos1], axis=1)
    pos_ref[...] = jnp.transpose(pos2, (1, 0)).astype(jnp.int32)

    wcol = lax.broadcasted_iota(jnp.int32, (T, 32), 1)
    w_ref[...] = jnp.where(wcol < 16, w0 / s, w1 / s)

    cumi = jnp.dot(pc, (same & (gi <= gj)).astype(jnp.float32),
                   preferred_element_type=jnp.float32)
    bstart = (lax.broadcasted_iota(jnp.int32, (NB_G, E), 0) * B)
    passed = (bstart.astype(jnp.float32) >= cumi).astype(jnp.float32)
    glo = jnp.sum(passed * (col[:NB_G, :] < EG), axis=-1, keepdims=True)
    ghi = jnp.sum(passed * (col[:NB_G, :] >= EG), axis=-1, keepdims=True)
    be0 = jnp.minimum(glo, EG - 1)
    be1 = jnp.minimum(ghi, EG - 1) + EG
    be2 = jnp.concatenate([be0, be1], axis=1)
    be_ref[...] = jnp.transpose(be2, (1, 0)).astype(jnp.int32)


def _routing(x, router_w):
    return pl.pallas_call(
        _routing_body,
        out_shape=(
            jax.ShapeDtypeStruct((G, T), jnp.int32),
            jax.ShapeDtypeStruct((T, 32), jnp.float32),
            jax.ShapeDtypeStruct((G, NB_G), jnp.int32),
        ),
    )(x, router_w)


def _make_mesh():
    return plsc.VectorSubcoreMesh(
        core_axis_name="c", subcore_axis_name="s",
        num_cores=NC, num_subcores=NS)


def _scatter_body(x_hbm, pos_hbm, out_hbm, i0_v, i1_v, rows_v, sem0, sem1):
    wid = lax.axis_index("s") * NC + lax.axis_index("c")
    base = wid * TPW
    pltpu.sync_copy(pos_hbm.at[0, pl.ds(base, TPW)], i0_v)
    pltpu.sync_copy(pos_hbm.at[1, pl.ds(base, TPW)], i1_v)
    pltpu.sync_copy(x_hbm.at[pl.ds(base, TPW)], rows_v)
    c0 = pltpu.async_copy(rows_v, out_hbm.at[i0_v], sem0)
    c1 = pltpu.async_copy(rows_v, out_hbm.at[i1_v], sem1)
    c0.wait()
    c1.wait()


def _dispatch(x, pos):
    return pl.kernel(
        _scatter_body,
        out_type=jax.ShapeDtypeStruct((ROWS, D), jnp.float32),
        mesh=_make_mesh(),
        scratch_types=[
            pltpu.VMEM((TPW,), jnp.int32),
            pltpu.VMEM((TPW,), jnp.int32),
            pltpu.VMEM((TPW, D), jnp.float32),
            pltpu.SemaphoreType.DMA,
            pltpu.SemaphoreType.DMA,
        ],
    )(x, pos)


def _ffn_body(be_ref, x_ref, wgu_ref, wd_ref, y_ref):
    e = be_ref[pl.program_id(0)]
    x = x_ref[...]
    gu = jnp.dot(x, wgu_ref[e], preferred_element_type=jnp.float32)
    g = gu[:, :F]
    u = gu[:, F:]
    h = g * (1.0 / (1.0 + jnp.exp(-g))) * u
    y_ref[...] = jnp.dot(h, wd_ref[e], preferred_element_type=jnp.float32)


def _ffn(block_expert, x_sorted, w_gate_up, w_down):
    grid_spec = pltpu.PrefetchScalarGridSpec(
        num_scalar_prefetch=1,
        grid=(NBT,),
        in_specs=[
            pl.BlockSpec((B, D), lambda b, be: (b, 0)),
            pl.BlockSpec((E, D, 2 * F), lambda b, be: (0, 0, 0)),
            pl.BlockSpec((E, F, D), lambda b, be: (0, 0, 0)),
        ],
        out_specs=pl.BlockSpec((B, D), lambda b, be: (b, 0)),
    )
    return pl.pallas_call(
        _ffn_body,
        grid_spec=grid_spec,
        out_shape=jax.ShapeDtypeStruct((ROWS, D), jnp.float32),
    )(block_expert, x_sorted, w_gate_up, w_down)


_CCH = TPW // 2


def _combine_body(y_hbm, pos_hbm, w_hbm, out_hbm,
                  i0_v, i1_v, w_v, b0_v, b1_v, gsem, wsem):
    wid = lax.axis_index("s") * NC + lax.axis_index("c")
    base = wid * TPW
    pltpu.sync_copy(pos_hbm.at[0, pl.ds(base, TPW)], i0_v)
    pltpu.sync_copy(pos_hbm.at[1, pl.ds(base, TPW)], i1_v)
    pltpu.sync_copy(w_hbm.at[pl.ds(base, TPW)], w_v)

    def chunk_copies(c):
        sl = pl.ds(c * _CCH, _CCH)
        return (pltpu.async_copy(y_hbm.at[i0_v.at[sl]],
                                 b0_v.at[sl], gsem),
                pltpu.async_copy(y_hbm.at[i1_v.at[sl]],
                                 b1_v.at[sl], gsem))

    def chunk_fma(c):
        def row_fma(r, carry):
            wa = w_v[r, pl.ds(0, 16)]
            wb = w_v[r, pl.ds(16, 16)]
            for j in range(D // 16):
                sl = (r, pl.ds(j * 16, 16))
                b0_v[sl] = b0_v[sl] * wa + b1_v[sl] * wb
            return carry
        lax.fori_loop(c * _CCH, (c + 1) * _CCH, row_fma, 0)

    g0a, g0b = chunk_copies(0)
    g1a, g1b = chunk_copies(1)
    g0a.wait()
    g0b.wait()
    chunk_fma(0)
    w0c = pltpu.async_copy(b0_v.at[pl.ds(0, _CCH)],
                           out_hbm.at[pl.ds(base, _CCH)], wsem)
    g1a.wait()
    g1b.wait()
    chunk_fma(1)
    pltpu.sync_copy(b0_v.at[pl.ds(_CCH, _CCH)],
                    out_hbm.at[pl.ds(base + _CCH, _CCH)])
    w0c.wait()


def _combine(y_sorted, pos, wrep):
    return pl.kernel(
        _combine_body,
        out_type=jax.ShapeDtypeStruct((T, D), jnp.float32),
        mesh=_make_mesh(),
        scratch_types=[
            pltpu.VMEM((TPW,), jnp.int32),
            pltpu.VMEM((TPW,), jnp.int32),
            pltpu.VMEM((TPW, 32), jnp.float32),
            pltpu.VMEM((TPW, D), jnp.float32),
            pltpu.VMEM((TPW, D), jnp.float32),
            pltpu.SemaphoreType.DMA,
            pltpu.SemaphoreType.DMA,
        ],
    )(y_sorted, pos, wrep)


def kernel(hidden_states, router_w, w_gate_up, w_down):
    x = hidden_states
    pos, wrep, bexp = _routing(x, router_w)
    block_expert = bexp.reshape(NBT)
    x_sorted = _dispatch(x, pos)
    y_sorted = _ffn(block_expert, x_sorted, w_gate_up, w_down)
    out = _combine(y_sorted, pos, wrep)
    return out.astype(hidden_states.dtype)


# speedup vs baseline: 1.1080x; 1.1080x over previous
import jax
import jax.numpy as jnp
from jax import lax
from jax.experimental import pallas as pl
from jax.experimental.pallas import tpu as pltpu
from jax.experimental.pallas import tpu_sc as plsc

E = 8
G = 2
EG = E // G
D = 768
F = 512
T = 2048
B = 256
NB_G = T // B + EG
NBT = G * NB_G
PG = NB_G * B
ROWS = G * PG
CHUNK = 256

NC, NS = 2, 16
NW = NC * NS
TPW = T // NW


def _routing_body(x_ref, rw_ref, pos_ref, w_ref, be_ref):
    x = x_ref[...]
    logits = jnp.dot(x, rw_ref[...], preferred_element_type=jnp.float32)
    m = jnp.max(logits, axis=-1, keepdims=True)
    ex = jnp.exp(logits - m)
    p = ex / jnp.sum(ex, axis=-1, keepdims=True)
    col = lax.broadcasted_iota(jnp.int32, (T, E), 1)
    g0 = col < EG
    neg = jnp.float32(-1.0)
    w0 = jnp.max(jnp.where(g0, p, neg), axis=-1, keepdims=True)
    w1 = jnp.max(jnp.where(g0, neg, p), axis=-1, keepdims=True)
    big = jnp.int32(E)
    e0 = jnp.min(jnp.where(g0 & (p == w0), col, big), axis=-1, keepdims=True)
    e1 = jnp.min(jnp.where((~g0) & (p == w1), col, big), axis=-1, keepdims=True)
    s = w0 + w1
    oh0 = (col == e0).astype(jnp.float32)
    oh1 = (col == e1).astype(jnp.float32)
    oh = oh0 + oh1

    r_ = lax.broadcasted_iota(jnp.int32, (CHUNK, CHUNK), 0)
    c_ = lax.broadcasted_iota(jnp.int32, (CHUNK, CHUNK), 1)
    tril = (r_ >= c_).astype(jnp.float32)
    carry = jnp.zeros((1, E), jnp.float32)
    rank_chunks = []
    for c in range(T // CHUNK):
        oh_c = oh[c * CHUNK:(c + 1) * CHUNK, :]
        cum_c = jnp.dot(tril, oh_c, preferred_element_type=jnp.float32)
        rank_chunks.append(cum_c + carry - oh_c)
        carry = carry + jnp.sum(oh_c, axis=0, keepdims=True)
    rank = jnp.concatenate(rank_chunks, axis=0)

    counts = carry
    pc = (jnp.floor(counts / B) +
          jnp.where(counts % B > 0, 1.0, 0.0)) * B
    gi = lax.broadcasted_iota(jnp.int32, (E, E), 0)
    gj = lax.broadcasted_iota(jnp.int32, (E, E), 1)
    same = gi // EG == gj // EG
    prefix = (same & (gi < gj)).astype(jnp.float32)
    off = jnp.dot(pc, prefix, preferred_element_type=jnp.float32)

    posf = jnp.sum(oh * (off + rank), axis=-1, keepdims=True)
    pos0 = jnp.sum(oh0 * (off + rank), axis=-1, keepdims=True)
    pos1 = posf - pos0 + PG
    pos2 = jnp.concatenate([pos0, pos1], axis=1)
    pos_ref[...] = jnp.transpose(pos2, (1, 0)).astype(jnp.int32)

    wcol = lax.broadcasted_iota(jnp.int32, (T, 32), 1)
    w_ref[...] = jnp.where(wcol < 16, w0 / s, w1 / s)

    cumi = jnp.dot(pc, (same & (gi <= gj)).astype(jnp.float32),
                   preferred_element_type=jnp.float32)
    bstart = (lax.broadcasted_iota(jnp.int32, (NB_G, E), 0) * B)
    passed = (bstart.astype(jnp.float32) >= cumi).astype(jnp.float32)
    lo = (col[:NB_G, :] < EG).astype(jnp.float32)
    glo = jnp.sum(passed * lo, axis=-1, keepdims=True)
    ghi = jnp.sum(passed * (1.0 - lo), axis=-1, keepdims=True)
    be0 = jnp.where(glo > EG - 1, -1.0, glo)
    be1 = jnp.where(ghi > EG - 1, -1.0, ghi + EG)
    be2 = jnp.concatenate([be0, be1], axis=1)
    be_ref[...] = jnp.transpose(be2, (1, 0)).astype(jnp.int32)


def _routing(x, router_w):
    return pl.pallas_call(
        _routing_body,
        out_shape=(
            jax.ShapeDtypeStruct((G, T), jnp.int32),
            jax.ShapeDtypeStruct((T, 32), jnp.float32),
            jax.ShapeDtypeStruct((G, NB_G), jnp.int32),
        ),
    )(x, router_w)


def _make_mesh():
    return plsc.VectorSubcoreMesh(
        core_axis_name="c", subcore_axis_name="s",
        num_cores=NC, num_subcores=NS)


def _scatter_body(x_hbm, pos_hbm, out_hbm, i0_v, i1_v, rows_v, sem0, sem1):
    wid = lax.axis_index("s") * NC + lax.axis_index("c")
    base = wid * TPW
    pltpu.sync_copy(pos_hbm.at[0, pl.ds(base, TPW)], i0_v)
    pltpu.sync_copy(pos_hbm.at[1, pl.ds(base, TPW)], i1_v)
    pltpu.sync_copy(x_hbm.at[pl.ds(base, TPW)], rows_v)
    c0 = pltpu.async_copy(rows_v, out_hbm.at[i0_v], sem0)
    c1 = pltpu.async_copy(rows_v, out_hbm.at[i1_v], sem1)
    c0.wait()
    c1.wait()


def _dispatch(x, pos):
    return pl.kernel(
        _scatter_body,
        out_type=jax.ShapeDtypeStruct((ROWS, D), jnp.float32),
        mesh=_make_mesh(),
        scratch_types=[
            pltpu.VMEM((TPW,), jnp.int32),
            pltpu.VMEM((TPW,), jnp.int32),
            pltpu.VMEM((TPW, D), jnp.float32),
            pltpu.SemaphoreType.DMA,
            pltpu.SemaphoreType.DMA,
        ],
    )(x, pos)


def _ffn_body(be_ref, x_ref, wgu_ref, wd_ref, y_ref):
    e = be_ref[pl.program_id(0)]

    @pl.when(e >= 0)
    def _compute():
        x = x_ref[...]
        gu = jnp.dot(x, wgu_ref[jnp.maximum(e, 0)],
                     preferred_element_type=jnp.float32)
        g = gu[:, :F]
        u = gu[:, F:]
        h = g * (1.0 / (1.0 + jnp.exp(-g))) * u
        y_ref[...] = jnp.dot(h, wd_ref[jnp.maximum(e, 0)],
                             preferred_element_type=jnp.float32)


def _ffn(block_expert, x_sorted, w_gate_up, w_down):
    grid_spec = pltpu.PrefetchScalarGridSpec(
        num_scalar_prefetch=1,
        grid=(NBT,),
        in_specs=[
            pl.BlockSpec((B, D), lambda b, be: (b, 0)),
            pl.BlockSpec((E, D, 2 * F), lambda b, be: (0, 0, 0)),
            pl.BlockSpec((E, F, D), lambda b, be: (0, 0, 0)),
        ],
        out_specs=pl.BlockSpec((B, D), lambda b, be: (b, 0)),
    )
    return pl.pallas_call(
        _ffn_body,
        grid_spec=grid_spec,
        out_shape=jax.ShapeDtypeStruct((ROWS, D), jnp.float32),
    )(block_expert, x_sorted, w_gate_up, w_down)


_CCH = TPW // 2


def _combine_body(y_hbm, pos_hbm, w_hbm, out_hbm,
                  i0_v, i1_v, w_v, b0_v, b1_v, gsem, wsem):
    wid = lax.axis_index("s") * NC + lax.axis_index("c")
    base = wid * TPW
    pltpu.sync_copy(pos_hbm.at[0, pl.ds(base, TPW)], i0_v)
    pltpu.sync_copy(pos_hbm.at[1, pl.ds(base, TPW)], i1_v)
    pltpu.sync_copy(w_hbm.at[pl.ds(base, TPW)], w_v)

    def chunk_copies(c):
        sl = pl.ds(c * _CCH, _CCH)
        return (pltpu.async_copy(y_hbm.at[i0_v.at[sl]],
                                 b0_v.at[sl], gsem),
                pltpu.async_copy(y_hbm.at[i1_v.at[sl]],
                                 b1_v.at[sl], gsem))

    def chunk_fma(c):
        def row_fma(r, carry):
            wa = w_v[r, pl.ds(0, 16)]
            wb = w_v[r, pl.ds(16, 16)]
            for j in range(D // 16):
                sl = (r, pl.ds(j * 16, 16))
                b0_v[sl] = b0_v[sl] * wa + b1_v[sl] * wb
            return carry
        lax.fori_loop(c * _CCH, (c + 1) * _CCH, row_fma, 0)

    g0a, g0b = chunk_copies(0)
    g1a, g1b = chunk_copies(1)
    g0a.wait()
    g0b.wait()
    chunk_fma(0)
    w0c = pltpu.async_copy(b0_v.at[pl.ds(0, _CCH)],
                           out_hbm.at[pl.ds(base, _CCH)], wsem)
    g1a.wait()
    g1b.wait()
    chunk_fma(1)
    pltpu.sync_copy(b0_v.at[pl.ds(_CCH, _CCH)],
                    out_hbm.at[pl.ds(base + _CCH, _CCH)])
    w0c.wait()


def _combine(y_sorted, pos, wrep):
    return pl.kernel(
        _combine_body,
        out_type=jax.ShapeDtypeStruct((T, D), jnp.float32),
        mesh=_make_mesh(),
        scratch_types=[
            pltpu.VMEM((TPW,), jnp.int32),
            pltpu.VMEM((TPW,), jnp.int32),
            pltpu.VMEM((TPW, 32), jnp.float32),
            pltpu.VMEM((TPW, D), jnp.float32),
            pltpu.VMEM((TPW, D), jnp.float32),
            pltpu.SemaphoreType.DMA,
            pltpu.SemaphoreType.DMA,
        ],
    )(y_sorted, pos, wrep)


def kernel(hidden_states, router_w, w_gate_up, w_down):
    x = hidden_states
    pos, wrep, bexp = _routing(x, router_w)
    block_expert = bexp.reshape(NBT)
    x_sorted = _dispatch(x, pos)
    y_sorted = _ffn(block_expert, x_sorted, w_gate_up, w_down)
    out = _combine(y_sorted, pos, wrep)
    return out.astype(hidden_states.dtype)
